# baseline (speedup 1.0000x reference)
import functools

import jax
import jax.numpy as jnp
from jax import lax
from jax.experimental import pallas as pl
from jax.experimental.pallas import tpu as pltpu
from jax.experimental.pallas import tpu_sc as plsc

N = 10000
E = 320000
H = 128
HP = H // 2
ET = 16
NFS = 128

NC = 2
NS = 16
NW = NC * NS
EPW = E // NW
K = 40
NB = EPW // K
NP = 10112
RPT = NP // NS

ROWS_TC = 1000
ROWS_E = 4000


def _leaky(x):
    return jnp.where(x >= 0, x, 0.1 * x)


def _dot(a, b):
    return jnp.dot(a, b, preferred_element_type=jnp.float32)


def _pack(x):
    u = lax.bitcast_convert_type(x.astype(jnp.bfloat16), jnp.uint16)
    u = u.astype(jnp.uint32)
    w = u[:, :HP] | (u[:, HP:] << 16)
    return lax.bitcast_convert_type(w, jnp.int32)


def _prologue_body(x_ref, w1_ref, b1_ref, w2_ref, b2_ref, wa_ref, wb_ref,
                   child_ref, ab_ref):
    h = _leaky(_leaky(_dot(x_ref[...], w1_ref[...]) + b1_ref[...]))
    c = _leaky(_dot(h, w2_ref[...]) + b2_ref[...])
    child_ref[...] = c
    ab_ref[...] = jnp.concatenate(
        [_pack(_dot(c, wa_ref[...])), _pack(_dot(c, wb_ref[...]))], axis=1)


def _mid_body(part_ref0, part_ref1, wa_ref, wb_ref, child_ref, ab_ref):
    c = part_ref0[0] + part_ref1[0]
    child_ref[...] = c
    ab_ref[...] = jnp.concatenate(
        [_pack(_dot(c, wa_ref[...])), _pack(_dot(c, wb_ref[...]))], axis=1)


def _edge_const_body(ef_ref, wc0_ref, be0_ref, wc1_ref, be1_ref,
                     c0_ref, c1_ref):
    ef = ef_ref[...]
    c0_ref[...] = _pack(_dot(ef, wc0_ref[...]) + be0_ref[...])
    c1_ref[...] = _pack(_dot(ef, wc1_ref[...]) + be1_ref[...])


def _final_body(part_ref0, part_ref1, c0_ref, c1_ref, ws0_ref, ws1_ref,
                ws2_ref, bs_ref, o_ref):
    c2 = part_ref0[0] + part_ref1[0]
    acc = _dot(c0_ref[...], ws0_ref[...])
    acc = acc + _dot(c1_ref[...], ws1_ref[...])
    acc = acc + _dot(c2, ws2_ref[...])
    o_ref[...] = _leaky(acc + bs_ref[...])


def _row_spec(rows, cols):
    return pl.BlockSpec((rows, cols), lambda i: (i, 0))


def _full_spec(rows, cols):
    return pl.BlockSpec((rows, cols), lambda i: (0, 0))


def _part_spec(half):
    return pl.BlockSpec((1, ROWS_TC, H), lambda i, h=half: (h, i, 0))


_prologue = pl.pallas_call(
    _prologue_body,
    grid=(N // ROWS_TC,),
    in_specs=[_row_spec(ROWS_TC, H), _full_spec(H, H), _full_spec(1, H),
              _full_spec(H, H), _full_spec(1, H), _full_spec(H, H),
              _full_spec(H, H)],
    out_specs=[_row_spec(ROWS_TC, H), _row_spec(ROWS_TC, H)],
    out_shape=[jax.ShapeDtypeStruct((N, H), jnp.float32),
               jax.ShapeDtypeStruct((N, H), jnp.int32)],
)

_mid = pl.pallas_call(
    _mid_body,
    grid=(N // ROWS_TC,),
    in_specs=[_part_spec(0), _part_spec(1),
              _full_spec(H, H), _full_spec(H, H)],
    out_specs=[_row_spec(ROWS_TC, H), _row_spec(ROWS_TC, H)],
    out_shape=[jax.ShapeDtypeStruct((N, H), jnp.float32),
               jax.ShapeDtypeStruct((N, H), jnp.int32)],
)

_edge_const = pl.pallas_call(
    _edge_const_body,
    grid=(E // ROWS_E,),
    in_specs=[_row_spec(ROWS_E, ET), _full_spec(ET, H), _full_spec(1, H),
              _full_spec(ET, H), _full_spec(1, H)],
    out_specs=[_row_spec(ROWS_E, HP)] * 2,
    out_shape=[jax.ShapeDtypeStruct((E, HP), jnp.int32)] * 2,
)

_final = pl.pallas_call(
    _final_body,
    grid=(N // ROWS_TC,),
    in_specs=[_part_spec(0), _part_spec(1),
              _row_spec(ROWS_TC, H), _row_spec(ROWS_TC, H)] +
             [_full_spec(H, H)] * 3 + [_full_spec(1, NFS)],
    out_specs=_row_spec(ROWS_TC, NFS),
    out_shape=jax.ShapeDtypeStruct((N, NFS), jnp.float32),
)


def _halves(u):
    lo = lax.bitcast_convert_type(u << 16, jnp.float32)
    hi = lax.bitcast_convert_type(u & jnp.int32(-65536), jnp.float32)
    return lo, hi


def _sc_body(ab_hbm, c_hbm, sd_hbm, zero_hbm, out_hbm, *refs):
    (sdv0, sdv1, scv0, scv1, gv0, gv1, cv0, cv1,
     mv0, mv1, acc, sg0, sg1, si0, si1, ss0, ss1) = refs
    sdv = (sdv0, sdv1)
    gv = (gv0, gv1)
    scv = (scv0, scv1)
    cv = (cv0, cv1)
    mv = (mv0, mv1)
    sg = (sg0, sg1)
    si = (si0, si1)
    ss = (ss0, ss1)

    cid = lax.axis_index("c")
    sid = lax.axis_index("s")
    wid = sid * NC + cid
    row0 = sid * RPT
    ebase = wid * EPW

    def issue_idx(b, p):
        pltpu.async_copy(sd_hbm.at[ebase // K + b], sdv[p], si[p])

    def wait_idx(p):
        pltpu.make_async_copy(sd_hbm.at[0], sdv[p], si[p]).wait()

    def issue_gathers(b, p):
        base = ebase + b * K
        pltpu.async_copy(ab_hbm.at[sdv[p]], gv[p], sg[p])
        pltpu.async_copy(c_hbm.at[pl.ds(base, K)], cv[p], sg[p])

    def wait_gathers(p):
        pltpu.make_async_copy(ab_hbm.at[sdv[p]], gv[p], sg[p]).wait()
        pltpu.make_async_copy(c_hbm.at[pl.ds(0, K)], cv[p], sg[p]).wait()

    def wait_scatter(p):
        pltpu.make_async_copy(mv[p], acc.at[scv[p]], ss[p]).wait()

    def process(b, p, start_next=True, drain=True, refill=True):
        if start_next:
            wait_idx(1 - p)
            issue_gathers(b + 1, 1 - p)
        wait_gathers(p)
        if drain:
            wait_scatter(p)
        for off in list(range(0, K - 15, 16)) + (
                [K - 16] if K % 16 else []):
            sl = pl.ds(off, 16)
            scv[p][sl] = sdv[p][sl]
        if refill:
            issue_idx(b + 2, p)

        @plsc.parallel_loop(0, K, unroll=4)
        def _row(r):
            for j in range(HP // 16):
                sj = pl.ds(j * 16, 16)
                sjh = pl.ds(HP + j * 16, 16)
                alo, ahi = _halves(gv[p][r, sj])
                blo, bhi = _halves(gv[p][K + r, sjh])
                clo, chi = _halves(cv[p][r, sj])
                mv[p][r, sj] = jnp.maximum(alo + blo + clo, 0.0)
                mv[p][r, sjh] = jnp.maximum(ahi + bhi + chi, 0.0)
        pltpu.async_copy(mv[p], acc.at[scv[p]], ss[p], add=True)

    pltpu.sync_copy(zero_hbm, acc.at[pl.ds(row0, RPT)])
    issue_idx(0, 0)
    wait_idx(0)
    issue_gathers(0, 0)
    issue_idx(1, 1)
    plsc.subcore_barrier()

    process(0, 0, drain=False)
    process(1, 1, drain=False)

    def pair(i, carry):
        b = 2 * i
        process(b, 0)
        process(b + 1, 1)
        return carry

    lax.fori_loop(1, NB // 2 - 1, pair, 0)
    process(NB - 2, 0, refill=False)
    process(NB - 1, 1, start_next=False, refill=False)
    wait_scatter(0)
    wait_scatter(1)
    plsc.subcore_barrier()
    pltpu.sync_copy(acc.at[pl.ds(row0, RPT)],
                    out_hbm.at[pl.ds(cid * NP + row0, RPT)])


_sc_pass = functools.partial(
    pl.kernel,
    out_type=jax.ShapeDtypeStruct((NC * NP, H), jnp.float32),
    mesh=plsc.VectorSubcoreMesh(core_axis_name="c", subcore_axis_name="s"),
    scratch_types=[
        pltpu.VMEM((2 * K,), jnp.int32),
        pltpu.VMEM((2 * K,), jnp.int32),
        pltpu.VMEM((K,), jnp.int32),
        pltpu.VMEM((K,), jnp.int32),
        pltpu.VMEM((2 * K, H), jnp.int32),
        pltpu.VMEM((2 * K, H), jnp.int32),
        pltpu.VMEM((K, HP), jnp.int32),
        pltpu.VMEM((K, HP), jnp.int32),
        pltpu.VMEM((K, H), jnp.float32),
        pltpu.VMEM((K, H), jnp.float32),
        pltpu.VMEM_SHARED((NP, H), jnp.float32),
        pltpu.SemaphoreType.DMA,
        pltpu.SemaphoreType.DMA,
        pltpu.SemaphoreType.DMA,
        pltpu.SemaphoreType.DMA,
        pltpu.SemaphoreType.DMA,
        pltpu.SemaphoreType.DMA,
    ],
)(_sc_body)


def kernel(child_feats, edge_indices, edge_type_onehot, W1, b1, W2, b2,
           We0, be0, We1, be1, Ws, bs):
    x = child_feats[0]
    sd = jnp.concatenate([edge_indices[0, :, 0].reshape(E // K, K),
                          edge_indices[0, :, 1].reshape(E // K, K)], axis=1)
    ef = edge_type_onehot[0]
    Wa0, Wb0, Wc0 = We0[:H], We0[H:2 * H], We0[2 * H:]
    Wa1, Wb1, Wc1 = We1[:H], We1[H:2 * H], We1[2 * H:]
    Ws0, Ws1, Ws2 = Ws[:H], Ws[H:2 * H], Ws[2 * H:]
    b1r = b1.reshape(1, H)
    b2r = b2.reshape(1, H)
    be0r = be0.reshape(1, H)
    be1r = be1.reshape(1, H)
    bsr = bs.reshape(1, NFS)
    zeros = jnp.zeros((RPT, H), jnp.float32)

    child0, AB0 = _prologue(x, W1, b1r, W2, b2r, Wa0, Wb0)
    C0, C1 = _edge_const(ef, Wc0, be0r, Wc1, be1r)
    part0 = _sc_pass(AB0, C0, sd, zeros).reshape(NC, NP, H)
    child1, AB1 = _mid(part0, part0, Wa1, Wb1)
    part1 = _sc_pass(AB1, C1, sd, zeros).reshape(NC, NP, H)
    return _final(part1, part1, child0, child1, Ws0, Ws1, Ws2, bsr)


# speedup vs baseline: 1.0196x; 1.0196x over previous
import functools

import jax
import jax.numpy as jnp
from jax import lax
from jax.experimental import pallas as pl
from jax.experimental.pallas import tpu as pltpu
from jax.experimental.pallas import tpu_sc as plsc

N = 10000
E = 320000
H = 128
HP = H // 2
ET = 16
NFS = 128

NC = 2
NS = 16
NW = NC * NS
EPW = E // NW
K = 40
NB = EPW // K
NP = 10112
RPT = NP // NS

ROWS_TC = 1000
ROWS_E = 4000


def _leaky(x):
    return jnp.where(x >= 0, x, 0.1 * x)


def _dot(a, b):
    return jnp.dot(a, b, preferred_element_type=jnp.float32)


def _pack(x):
    u = lax.bitcast_convert_type(x.astype(jnp.bfloat16), jnp.uint16)
    u = u.astype(jnp.uint32)
    w = u[:, :HP] | (u[:, HP:] << 16)
    return lax.bitcast_convert_type(w, jnp.int32)


def _prologue_body(x_ref, w1_ref, b1_ref, w2_ref, b2_ref, wa_ref, wb_ref,
                   child_ref, ab_ref):
    h = _leaky(_leaky(_dot(x_ref[...], w1_ref[...]) + b1_ref[...]))
    c = _leaky(_dot(h, w2_ref[...]) + b2_ref[...])
    child_ref[...] = c
    ab_ref[...] = jnp.concatenate(
        [_pack(_dot(c, wa_ref[...])), _pack(_dot(c, wb_ref[...]))], axis=1)


def _mid_body(part_ref0, part_ref1, wa_ref, wb_ref, child_ref, ab_ref):
    c = part_ref0[0] + part_ref1[0]
    child_ref[...] = c
    ab_ref[...] = jnp.concatenate(
        [_pack(_dot(c, wa_ref[...])), _pack(_dot(c, wb_ref[...]))], axis=1)


def _edge_const_body(ef_ref, wc0_ref, be0_ref, wc1_ref, be1_ref,
                     c0_ref, c1_ref):
    ef = ef_ref[...]
    c0_ref[...] = _pack(_dot(ef, wc0_ref[...]) + be0_ref[...])
    c1_ref[...] = _pack(_dot(ef, wc1_ref[...]) + be1_ref[...])


def _final_body(part_ref0, part_ref1, c0_ref, c1_ref, ws0_ref, ws1_ref,
                ws2_ref, bs_ref, o_ref):
    c2 = part_ref0[0] + part_ref1[0]
    acc = _dot(c0_ref[...], ws0_ref[...])
    acc = acc + _dot(c1_ref[...], ws1_ref[...])
    acc = acc + _dot(c2, ws2_ref[...])
    o_ref[...] = _leaky(acc + bs_ref[...])


def _row_spec(rows, cols):
    return pl.BlockSpec((rows, cols), lambda i: (i, 0))


def _full_spec(rows, cols):
    return pl.BlockSpec((rows, cols), lambda i: (0, 0))


def _part_spec(half):
    return pl.BlockSpec((1, ROWS_TC, H), lambda i, h=half: (h, i, 0))


_prologue = pl.pallas_call(
    _prologue_body,
    grid=(N // ROWS_TC,),
    in_specs=[_row_spec(ROWS_TC, H), _full_spec(H, H), _full_spec(1, H),
              _full_spec(H, H), _full_spec(1, H), _full_spec(H, H),
              _full_spec(H, H)],
    out_specs=[_row_spec(ROWS_TC, H), _row_spec(ROWS_TC, H)],
    out_shape=[jax.ShapeDtypeStruct((N, H), jnp.float32),
               jax.ShapeDtypeStruct((N, H), jnp.int32)],
)

_mid = pl.pallas_call(
    _mid_body,
    grid=(N // ROWS_TC,),
    in_specs=[_part_spec(0), _part_spec(1),
              _full_spec(H, H), _full_spec(H, H)],
    out_specs=[_row_spec(ROWS_TC, H), _row_spec(ROWS_TC, H)],
    out_shape=[jax.ShapeDtypeStruct((N, H), jnp.float32),
               jax.ShapeDtypeStruct((N, H), jnp.int32)],
)

_edge_const = pl.pallas_call(
    _edge_const_body,
    grid=(E // ROWS_E,),
    in_specs=[_row_spec(ROWS_E, ET), _full_spec(ET, H), _full_spec(1, H),
              _full_spec(ET, H), _full_spec(1, H)],
    out_specs=[_row_spec(ROWS_E, HP)] * 2,
    out_shape=[jax.ShapeDtypeStruct((E, HP), jnp.int32)] * 2,
)

_final = pl.pallas_call(
    _final_body,
    grid=(N // ROWS_TC,),
    in_specs=[_part_spec(0), _part_spec(1),
              _row_spec(ROWS_TC, H), _row_spec(ROWS_TC, H)] +
             [_full_spec(H, H)] * 3 + [_full_spec(1, NFS)],
    out_specs=_row_spec(ROWS_TC, NFS),
    out_shape=jax.ShapeDtypeStruct((N, NFS), jnp.float32),
)


def _halves(u):
    lo = lax.bitcast_convert_type(u << 16, jnp.float32)
    hi = lax.bitcast_convert_type(u & jnp.int32(-65536), jnp.float32)
    return lo, hi


def _sc_body(ab_hbm, c_hbm, src_hbm, dst_hbm, zero_hbm, out_hbm, *refs):
    (srcv0, srcv1, dstv0, dstv1, scv0, scv1, av0, av1, bv0, bv1, cv0, cv1,
     mv0, mv1, acc, sg0, sg1, si0, si1, ss0, ss1) = refs
    srcv = (srcv0, srcv1)
    dstv = (dstv0, dstv1)
    scv = (scv0, scv1)
    av = (av0, av1)
    bv = (bv0, bv1)
    cv = (cv0, cv1)
    mv = (mv0, mv1)
    sg = (sg0, sg1)
    si = (si0, si1)
    ss = (ss0, ss1)

    cid = lax.axis_index("c")
    sid = lax.axis_index("s")
    wid = sid * NC + cid
    row0 = sid * RPT
    ebase = wid * EPW

    def issue_idx(b, p):
        base = ebase + b * K
        pltpu.async_copy(src_hbm.at[pl.ds(base, K)], srcv[p], si[p])
        pltpu.async_copy(dst_hbm.at[pl.ds(base, K)], dstv[p], si[p])

    def wait_idx(p):
        pltpu.make_async_copy(src_hbm.at[pl.ds(0, K)], srcv[p], si[p]).wait()
        pltpu.make_async_copy(dst_hbm.at[pl.ds(0, K)], dstv[p], si[p]).wait()

    def issue_gathers(b, p):
        base = ebase + b * K
        pltpu.async_copy(ab_hbm.at[srcv[p]], av[p], sg[p])
        pltpu.async_copy(ab_hbm.at[dstv[p]], bv[p], sg[p])
        pltpu.async_copy(c_hbm.at[pl.ds(base, K)], cv[p], sg[p])

    def wait_gathers(p):
        pltpu.make_async_copy(ab_hbm.at[srcv[p]], av[p], sg[p]).wait()
        pltpu.make_async_copy(ab_hbm.at[dstv[p]], bv[p], sg[p]).wait()
        pltpu.make_async_copy(c_hbm.at[pl.ds(0, K)], cv[p], sg[p]).wait()

    def wait_scatter(p):
        pltpu.make_async_copy(mv[p], acc.at[scv[p]], ss[p]).wait()

    def process(b, p, start_next=True, drain=True, refill=True):
        if start_next:
            wait_idx(1 - p)
            issue_gathers(b + 1, 1 - p)
        wait_gathers(p)
        if drain:
            wait_scatter(p)
        for off in list(range(0, K - 15, 16)) + (
                [K - 16] if K % 16 else []):
            sl = pl.ds(off, 16)
            scv[p][sl] = srcv[p][sl]
        if refill:
            issue_idx(b + 2, p)

        @plsc.parallel_loop(0, K, unroll=4)
        def _row(r):
            for j in range(HP // 16):
                sj = pl.ds(j * 16, 16)
                sjh = pl.ds(HP + j * 16, 16)
                alo, ahi = _halves(av[p][r, sj])
                blo, bhi = _halves(bv[p][r, sjh])
                clo, chi = _halves(cv[p][r, sj])
                mv[p][r, sj] = jnp.maximum(alo + blo + clo, 0.0)
                mv[p][r, sjh] = jnp.maximum(ahi + bhi + chi, 0.0)
        pltpu.async_copy(mv[p], acc.at[scv[p]], ss[p], add=True)

    pltpu.sync_copy(zero_hbm, acc.at[pl.ds(row0, RPT)])
    issue_idx(0, 0)
    wait_idx(0)
    issue_gathers(0, 0)
    issue_idx(1, 1)
    plsc.subcore_barrier()

    process(0, 0, drain=False)
    process(1, 1, drain=False)

    def pair(i, carry):
        b = 2 * i
        process(b, 0)
        process(b + 1, 1)
        return carry

    lax.fori_loop(1, NB // 2 - 1, pair, 0)
    process(NB - 2, 0, refill=False)
    process(NB - 1, 1, start_next=False, refill=False)
    wait_scatter(0)
    wait_scatter(1)
    plsc.subcore_barrier()
    pltpu.sync_copy(acc.at[pl.ds(row0, RPT)],
                    out_hbm.at[pl.ds(cid * NP + row0, RPT)])


_sc_pass = functools.partial(
    pl.kernel,
    out_type=jax.ShapeDtypeStruct((NC * NP, H), jnp.float32),
    mesh=plsc.VectorSubcoreMesh(core_axis_name="c", subcore_axis_name="s"),
    scratch_types=[
        pltpu.VMEM((K,), jnp.int32),
        pltpu.VMEM((K,), jnp.int32),
        pltpu.VMEM((K,), jnp.int32),
        pltpu.VMEM((K,), jnp.int32),
        pltpu.VMEM((K,), jnp.int32),
        pltpu.VMEM((K,), jnp.int32),
        pltpu.VMEM((K, H), jnp.int32),
        pltpu.VMEM((K, H), jnp.int32),
        pltpu.VMEM((K, H), jnp.int32),
        pltpu.VMEM((K, H), jnp.int32),
        pltpu.VMEM((K, HP), jnp.int32),
        pltpu.VMEM((K, HP), jnp.int32),
        pltpu.VMEM((K, H), jnp.float32),
        pltpu.VMEM((K, H), jnp.float32),
        pltpu.VMEM_SHARED((NP, H), jnp.float32),
        pltpu.SemaphoreType.DMA,
        pltpu.SemaphoreType.DMA,
        pltpu.SemaphoreType.DMA,
        pltpu.SemaphoreType.DMA,
        pltpu.SemaphoreType.DMA,
        pltpu.SemaphoreType.DMA,
    ],
)(_sc_body)


def kernel(child_feats, edge_indices, edge_type_onehot, W1, b1, W2, b2,
           We0, be0, We1, be1, Ws, bs):
    x = child_feats[0]
    src = edge_indices[0, :, 0]
    dst = edge_indices[0, :, 1]
    ef = edge_type_onehot[0]
    Wa0, Wb0, Wc0 = We0[:H], We0[H:2 * H], We0[2 * H:]
    Wa1, Wb1, Wc1 = We1[:H], We1[H:2 * H], We1[2 * H:]
    Ws0, Ws1, Ws2 = Ws[:H], Ws[H:2 * H], Ws[2 * H:]
    b1r = b1.reshape(1, H)
    b2r = b2.reshape(1, H)
    be0r = be0.reshape(1, H)
    be1r = be1.reshape(1, H)
    bsr = bs.reshape(1, NFS)
    zeros = jnp.zeros((RPT, H), jnp.float32)

    child0, AB0 = _prologue(x, W1, b1r, W2, b2r, Wa0, Wb0)
    C0, C1 = _edge_const(ef, Wc0, be0r, Wc1, be1r)
    part0 = _sc_pass(AB0, C0, src, dst, zeros).reshape(NC, NP, H)
    child1, AB1 = _mid(part0, part0, Wa1, Wb1)
    part1 = _sc_pass(AB1, C1, src, dst, zeros).reshape(NC, NP, H)
    return _final(part1, part1, child0, child1, Ws0, Ws1, Ws2, bsr)
